# Initial kernel scaffold; baseline (speedup 1.0000x reference)
import jax
import jax.numpy as jnp
from jax.experimental import pallas as pl


def kernel(x, W, b, codebook):
    raise NotImplementedError("write your pallas kernel here")


# speedup vs baseline: 4.2014x; 4.2014x over previous
import jax
import jax.numpy as jnp
from jax.experimental import pallas as pl
from jax.experimental.pallas import tpu as pltpu

_B, _T, _C = 4, 1024, 512
_G, _V = 2, 320
_GV = _G * _V
_D = 128
_N = _B * _T
_BLK = 512
_GRID = _N // _BLK
_MAX_TEMP = 2.0


def _vq_kernel(x_ref, wt_ref, b_ref, cb2_ref, out_ref, ppl_ref, acc_ref):
    i = pl.program_id(0)

    @pl.when(i == 0)
    def _init():
        acc_ref[...] = jnp.zeros_like(acc_ref)

    logits = jnp.dot(x_ref[...], wt_ref[...],
                     preferred_element_type=jnp.float32) + b_ref[...]

    cols = jax.lax.broadcasted_iota(jnp.int32, (_BLK, _GV), 1)
    g0 = cols < _V
    neg = jnp.float32(-jnp.inf)
    l0 = jnp.where(g0, logits, neg)
    l1 = jnp.where(g0, neg, logits)
    m0 = jnp.max(l0, axis=1, keepdims=True)
    m1 = jnp.max(l1, axis=1, keepdims=True)
    idx0 = jnp.min(jnp.where(l0 == m0, cols, _GV), axis=1, keepdims=True)
    idx1 = jnp.min(jnp.where(l1 == m1, cols, _GV), axis=1, keepdims=True)
    onehot = ((cols == idx0) | (cols == idx1)).astype(jnp.float32)
    out_ref[...] = jnp.dot(onehot, cb2_ref[...],
                           preferred_element_type=jnp.float32)

    m = jnp.where(g0, m0, m1)
    e = jnp.exp(logits - m)
    s0 = jnp.sum(jnp.where(g0, e, 0.0), axis=1, keepdims=True)
    s1 = jnp.sum(jnp.where(g0, 0.0, e), axis=1, keepdims=True)
    probs = e / jnp.where(g0, s0, s1)
    acc_ref[...] += jnp.sum(probs, axis=0, keepdims=True)

    @pl.when(i == _GRID - 1)
    def _epilogue():
        avg = acc_ref[...] / jnp.float32(_N)
        plogp = avg * jnp.log(avg + jnp.float32(1e-7))
        c1 = jax.lax.broadcasted_iota(jnp.int32, (1, _GV), 1)
        in_g0 = c1 < _V
        ent0 = -jnp.sum(jnp.where(in_g0, plogp, 0.0))
        ent1 = -jnp.sum(jnp.where(in_g0, 0.0, plogp))
        ppl = jnp.exp(ent0) + jnp.exp(ent1)
        val = (jnp.float32(_GV) - ppl) / jnp.float32(_GV)
        ppl_ref[...] = jnp.full((1, 1), val, jnp.float32)


def kernel(x, W, b, codebook):
    flat = x.reshape(_N, _C)
    wt = W.T
    b2 = b.reshape(1, _GV)
    cb = codebook.reshape(_G, _V, _D)
    cb2 = jnp.zeros((_GV, _G * _D), jnp.float32)
    cb2 = cb2.at[:_V, :_D].set(cb[0]).at[_V:, _D:].set(cb[1])

    out_flat, ppl = pl.pallas_call(
        _vq_kernel,
        grid=(_GRID,),
        in_specs=[
            pl.BlockSpec((_BLK, _C), lambda i: (i, 0)),
            pl.BlockSpec((_C, _GV), lambda i: (0, 0)),
            pl.BlockSpec((1, _GV), lambda i: (0, 0)),
            pl.BlockSpec((_GV, _G * _D), lambda i: (0, 0)),
        ],
        out_specs=[
            pl.BlockSpec((_BLK, _G * _D), lambda i: (i, 0)),
            pl.BlockSpec((1, 1), lambda i: (0, 0)),
        ],
        out_shape=[
            jax.ShapeDtypeStruct((_N, _G * _D), jnp.float32),
            jax.ShapeDtypeStruct((1, 1), jnp.float32),
        ],
        scratch_shapes=[pltpu.VMEM((1, _GV), jnp.float32)],
    )(flat, wt, b2, cb2)

    out = out_flat.reshape(_B, _T, _G * _D)
    return (out, ppl.reshape(()), jnp.float32(_MAX_TEMP))
